# baseline (speedup 1.0000x reference)
import functools

import jax
import jax.numpy as jnp
from jax import lax
from jax.experimental import pallas as pl
from jax.experimental.pallas import tpu as pltpu
from jax.experimental.pallas import tpu_sc as plsc

NC = 2
NS = 16
B = 128

_MESH = plsc.VectorSubcoreMesh(core_axis_name="c", subcore_axis_name="s")
_SC_PARAMS = pltpu.CompilerParams(use_tc_tiling_on_sc=False)


def _ceil_to(x, m):
    return -(-x // m) * m


def _deg_body(nblk, rpt, dst_h, out_h, idx_d, ones_v, zero_v, acc_sh, *sems):
    c = lax.axis_index("c")
    s = lax.axis_index("s")
    tile = c * NS + s
    for j in range(B // 16):
        ones_v[pl.ds(16 * j, 16)] = jnp.ones((16,), jnp.float32)
    @pl.loop(0, rpt // 16)
    def _(j):
        zero_v[pl.ds(16 * j, 16)] = jnp.zeros((16,), jnp.float32)
    pltpu.sync_copy(zero_v, acc_sh.at[pl.ds(s * rpt, rpt)])
    pltpu.sync_copy(dst_h.at[pl.ds(tile * nblk, nblk)], idx_d)
    plsc.subcore_barrier()

    @pl.loop(0, nblk // NBUF)
    def _(g):
        descs = [pltpu.async_copy(ones_v,
                                  acc_sh.at[idx_d.at[g * NBUF + b]],
                                  sems[b], add=True)
                 for b in range(NBUF)]
        for d in descs:
            d.wait()

    plsc.subcore_barrier()
    pltpu.sync_copy(acc_sh.at[pl.ds(s * rpt, rpt)],
                    out_h.at[c, pl.ds(s * rpt, rpt)])


def _degree(dst_r, np_, nblk):
    rpt = np_ // NS
    body = functools.partial(_deg_body, nblk, rpt)
    return pl.kernel(
        body,
        out_type=jax.ShapeDtypeStruct((NC, np_), jnp.float32),
        mesh=_MESH,
        scratch_types=[
            pltpu.VMEM((nblk, B), jnp.int32),
            pltpu.VMEM((B,), jnp.float32),
            pltpu.VMEM((rpt,), jnp.float32),
            pltpu.VMEM_SHARED((np_,), jnp.float32),
        ] + [pltpu.SemaphoreType.DMA] * NBUF,
        compiler_params=_SC_PARAMS,
    )(dst_r)


NBUF = 4


def _edge_loop(ch, nblk, row_base, tab_sh, tab_hbm, src_h, dst_h,
               idx_s, idx_d, rows, acc_sh, gsems, ssems):
    dummy_h = tab_hbm.at[pl.ds(0, B)]

    def gsrc(b):
        return tab_sh

    @pl.loop(0, nblk // ch)
    def _(cc):
        row0 = row_base + cc * ch
        pltpu.sync_copy(src_h.at[pl.ds(row0, ch)], idx_s)
        pltpu.sync_copy(dst_h.at[pl.ds(row0, ch)], idx_d)
        for b in range(NBUF):
            pltpu.async_copy(gsrc(b).at[idx_s.at[b]], rows.at[b], gsems[b])

        @pl.loop(0, ch // NBUF)
        def _(g):
            j0 = g * NBUF
            descs = []
            for b in range(NBUF):
                pltpu.make_async_copy(dummy_h, rows.at[b], gsems[b]).wait()
                descs.append(pltpu.async_copy(
                    rows.at[b], acc_sh.at[idx_d.at[j0 + b]], ssems[b],
                    add=True))
            for b in range(NBUF):
                descs[b].wait()

                @pl.when(j0 + NBUF + b < ch)
                def _():
                    pltpu.async_copy(gsrc(b).at[idx_s.at[j0 + NBUF + b]],
                                     rows.at[b], gsems[b])


def _edge_fs_body(tbt, rpt, ch, table_h, src_h, dst_h, out_h,
                  idx_s, idx_d, rows, tab_sh, acc_sh, *sems):
    c = lax.axis_index("c")
    s = lax.axis_index("s")
    sl = pl.ds(s * rpt, rpt)
    d1 = pltpu.async_copy(table_h.at[c, sl], tab_sh.at[sl], sems[0])
    d2 = pltpu.async_copy(table_h.at[c, sl], acc_sh.at[sl], sems[1])
    d1.wait()
    d2.wait()
    plsc.subcore_barrier()
    _edge_loop(ch, tbt, s * tbt, tab_sh, table_h.at[c], src_h, dst_h,
               idx_s, idx_d, rows, acc_sh, sems[:NBUF], sems[NBUF:])
    plsc.subcore_barrier()
    pltpu.sync_copy(acc_sh.at[sl], out_h.at[c, sl])


def _edge_es_body(nblk0, nblk1, rpt, ch, table_h, src_h, dst_h, out_h,
                  idx_s, idx_d, rows, tab_sh, acc_sh, *sems):
    c = lax.axis_index("c")
    s = lax.axis_index("s")
    sl = pl.ds(s * rpt, rpt)
    d1 = pltpu.async_copy(table_h.at[sl], tab_sh.at[sl], sems[0])
    d2 = pltpu.async_copy(table_h.at[sl], acc_sh.at[sl], sems[1])
    d1.wait()
    d2.wait()
    plsc.subcore_barrier()

    args = (tab_sh, table_h, src_h, dst_h, idx_s, idx_d,
            rows, acc_sh, sems[:NBUF], sems[NBUF:])

    @pl.when(c == 0)
    def _():
        _edge_loop(ch, nblk0, s * nblk0, *args)

    @pl.when(c == 1)
    def _():
        _edge_loop(ch, nblk1, NS * nblk0 + s * nblk1, *args)

    plsc.subcore_barrier()
    pltpu.sync_copy(acc_sh.at[sl], out_h.at[c, sl])


def _edge_scratch(ch, np_, f):
    return [
        pltpu.VMEM((ch, B), jnp.int32),
        pltpu.VMEM((ch, B), jnp.int32),
        pltpu.VMEM((NBUF, B, f), jnp.float32),
        pltpu.VMEM_SHARED((np_, f), jnp.float32),
        pltpu.VMEM_SHARED((np_, f), jnp.float32),
    ] + [pltpu.SemaphoreType.DMA] * (2 * NBUF)


def _edge_pass_fs(table2, src_r, dst_r, np_, f2, tbt):
    rpt = np_ // NS
    ch = 16
    body = functools.partial(_edge_fs_body, tbt, rpt, ch)
    return pl.kernel(
        body,
        out_type=jax.ShapeDtypeStruct((NC, np_, f2), jnp.float32),
        mesh=_MESH,
        scratch_types=_edge_scratch(ch, np_, f2),
        compiler_params=_SC_PARAMS,
    )(table2, src_r, dst_r)


def _edge_pass_es(table, src_r, dst_r, np_, f, nblk0, nblk1):
    rpt = np_ // NS
    ch = 16
    body = functools.partial(_edge_es_body, nblk0, nblk1, rpt, ch)
    return pl.kernel(
        body,
        out_type=jax.ShapeDtypeStruct((NC, np_, f), jnp.float32),
        mesh=_MESH,
        scratch_types=_edge_scratch(ch, np_, f),
        compiler_params=_SC_PARAMS,
    )(table, src_r, dst_r)


def _b_body(f2, x_ref, w_ref, degp_ref, out_ref):
    deg = degp_ref[0, :] + degp_ref[1, :] + 1.0
    dinv = lax.rsqrt(deg)
    h = jnp.dot(x_ref[...], w_ref[...], preferred_element_type=jnp.float32)
    hp = h * dinv[:, None]
    out_ref[0] = hp[:, :f2]
    out_ref[1] = hp[:, f2:]


def _d_body(acc_ref, degp_ref, b1_ref, w2_ref, out_ref):
    deg = degp_ref[0, :] + degp_ref[1, :] + 1.0
    dinv = lax.rsqrt(deg)
    ssum = jnp.concatenate([acc_ref[0], acc_ref[1]], axis=1)
    a1 = jnp.maximum(ssum * dinv[:, None] + b1_ref[...], 0.0)
    h2 = jnp.dot(a1, w2_ref[...], preferred_element_type=jnp.float32)
    out_ref[...] = h2 * dinv[:, None]


def _f_body(acc_ref, hp_ref, degt_ref, b2_ref, out_ref):
    d = degt_ref[...]
    dinv = lax.rsqrt(d[:, 0] + d[:, 1] + 1.0)
    out_ref[...] = (acc_ref[0] + acc_ref[1] - hp_ref[...]) * dinv[:, None] \
        + b2_ref[...]


def kernel(x, edge_index, W1, b1, W2, b2):
    n, d_in = x.shape
    hid = W1.shape[1]
    n_act = W2.shape[1]
    e = edge_index.shape[1]

    np_ = _ceil_to(n + 1, NS * 40)
    ep = _ceil_to(e, NC * NS * B * 8)
    nblk = ep // (NC * NS * B)
    tbt = ep // (NS * B)
    nblk0 = tbt // 2
    nblk1 = tbt - nblk0
    f2 = hid // NC
    rb = np_ // 16

    pad_idx = jnp.full((2, ep - e), n, dtype=edge_index.dtype)
    ei = jnp.concatenate([edge_index, pad_idx], axis=1)
    src_r = ei[0].reshape(NC * NS * nblk, B)
    dst_r = ei[1].reshape(NC * NS * nblk, B)
    xp = jnp.pad(x, ((0, np_ - n), (0, 0)))

    degp = _degree(dst_r, np_, nblk)

    h1t = pl.pallas_call(
        functools.partial(_b_body, f2),
        grid=(np_ // rb,),
        in_specs=[
            pl.BlockSpec((rb, d_in), lambda i: (i, 0)),
            pl.BlockSpec((d_in, hid), lambda i: (0, 0)),
            pl.BlockSpec((NC, rb), lambda i: (0, i)),
        ],
        out_specs=pl.BlockSpec((NC, rb, f2), lambda i: (0, i, 0)),
        out_shape=jax.ShapeDtypeStruct((NC, np_, f2), jnp.float32),
    )(xp, W1, degp)

    acc1 = _edge_pass_fs(h1t, src_r, dst_r, np_, f2, tbt)

    h2p = pl.pallas_call(
        _d_body,
        grid=(np_ // rb,),
        in_specs=[
            pl.BlockSpec((NC, rb, f2), lambda i: (0, i, 0)),
            pl.BlockSpec((NC, rb), lambda i: (0, i)),
            pl.BlockSpec((1, hid), lambda i: (0, 0)),
            pl.BlockSpec((hid, n_act), lambda i: (0, 0)),
        ],
        out_specs=pl.BlockSpec((rb, n_act), lambda i: (i, 0)),
        out_shape=jax.ShapeDtypeStruct((np_, n_act), jnp.float32),
    )(acc1, degp, b1.reshape(1, hid), W2)

    acc2 = _edge_pass_es(h2p, src_r, dst_r, np_, n_act, nblk0, nblk1)

    rbf = 400
    out = pl.pallas_call(
        _f_body,
        grid=(n // rbf,),
        in_specs=[
            pl.BlockSpec((NC, rbf, n_act), lambda i: (0, i, 0)),
            pl.BlockSpec((rbf, n_act), lambda i: (i, 0)),
            pl.BlockSpec((rbf, NC), lambda i: (i, 0)),
            pl.BlockSpec((1, n_act), lambda i: (0, 0)),
        ],
        out_specs=pl.BlockSpec((rbf, n_act), lambda i: (i, 0)),
        out_shape=jax.ShapeDtypeStruct((n, n_act), jnp.float32),
    )(acc2, h2p, degp.T, b2.reshape(1, n_act))

    return out


# speedup vs baseline: 1.0266x; 1.0266x over previous
import functools

import jax
import jax.numpy as jnp
from jax import lax
from jax.experimental import pallas as pl
from jax.experimental.pallas import tpu as pltpu
from jax.experimental.pallas import tpu_sc as plsc

NC = 2
NS = 16
B = 256

_MESH = plsc.VectorSubcoreMesh(core_axis_name="c", subcore_axis_name="s")
_SC_PARAMS = pltpu.CompilerParams(use_tc_tiling_on_sc=False)


def _ceil_to(x, m):
    return -(-x // m) * m


def _deg_body(nblk, rpt, dst_h, out_h, idx_d, ones_v, zero_v, acc_sh, *sems):
    c = lax.axis_index("c")
    s = lax.axis_index("s")
    tile = c * NS + s
    for j in range(B // 16):
        ones_v[pl.ds(16 * j, 16)] = jnp.ones((16,), jnp.float32)
    @pl.loop(0, rpt // 16)
    def _(j):
        zero_v[pl.ds(16 * j, 16)] = jnp.zeros((16,), jnp.float32)
    pltpu.sync_copy(zero_v, acc_sh.at[pl.ds(s * rpt, rpt)])
    pltpu.sync_copy(dst_h.at[pl.ds(tile * nblk, nblk)], idx_d)
    plsc.subcore_barrier()

    @pl.loop(0, nblk // NBUF)
    def _(g):
        descs = [pltpu.async_copy(ones_v,
                                  acc_sh.at[idx_d.at[g * NBUF + b]],
                                  sems[b], add=True)
                 for b in range(NBUF)]
        for d in descs:
            d.wait()

    plsc.subcore_barrier()
    pltpu.sync_copy(acc_sh.at[pl.ds(s * rpt, rpt)],
                    out_h.at[c, pl.ds(s * rpt, rpt)])


def _degree(dst_r, np_, nblk):
    rpt = np_ // NS
    body = functools.partial(_deg_body, nblk, rpt)
    return pl.kernel(
        body,
        out_type=jax.ShapeDtypeStruct((NC, np_), jnp.float32),
        mesh=_MESH,
        scratch_types=[
            pltpu.VMEM((nblk, B), jnp.int32),
            pltpu.VMEM((B,), jnp.float32),
            pltpu.VMEM((rpt,), jnp.float32),
            pltpu.VMEM_SHARED((np_,), jnp.float32),
        ] + [pltpu.SemaphoreType.DMA] * NBUF,
        compiler_params=_SC_PARAMS,
    )(dst_r)


NBUF = 2


def _edge_loop(ch, nblk, row_base, tab_sh, tab_hbm, src_h, dst_h,
               idx_s, idx_d, rows, acc_sh, gsems, ssems):
    dummy_h = tab_hbm.at[pl.ds(0, B)]

    def gsrc(b):
        return tab_sh

    @pl.loop(0, nblk // ch)
    def _(cc):
        row0 = row_base + cc * ch
        pltpu.sync_copy(src_h.at[pl.ds(row0, ch)], idx_s)
        pltpu.sync_copy(dst_h.at[pl.ds(row0, ch)], idx_d)
        for b in range(NBUF):
            pltpu.async_copy(gsrc(b).at[idx_s.at[b]], rows.at[b], gsems[b])

        @pl.loop(0, ch // NBUF)
        def _(g):
            j0 = g * NBUF
            descs = []
            for b in range(NBUF):
                pltpu.make_async_copy(dummy_h, rows.at[b], gsems[b]).wait()
                descs.append(pltpu.async_copy(
                    rows.at[b], acc_sh.at[idx_d.at[j0 + b]], ssems[b],
                    add=True))
            for b in range(NBUF):
                descs[b].wait()

                @pl.when(j0 + NBUF + b < ch)
                def _():
                    pltpu.async_copy(gsrc(b).at[idx_s.at[j0 + NBUF + b]],
                                     rows.at[b], gsems[b])


def _edge_fs_body(tbt, rpt, ch, table_h, src_h, dst_h, out_h,
                  idx_s, idx_d, rows, tab_sh, acc_sh, *sems):
    c = lax.axis_index("c")
    s = lax.axis_index("s")
    sl = pl.ds(s * rpt, rpt)
    d1 = pltpu.async_copy(table_h.at[c, sl], tab_sh.at[sl], sems[0])
    d2 = pltpu.async_copy(table_h.at[c, sl], acc_sh.at[sl], sems[1])
    d1.wait()
    d2.wait()
    plsc.subcore_barrier()
    _edge_loop(ch, tbt, s * tbt, tab_sh, table_h.at[c], src_h, dst_h,
               idx_s, idx_d, rows, acc_sh, sems[:NBUF], sems[NBUF:])
    plsc.subcore_barrier()
    pltpu.sync_copy(acc_sh.at[sl], out_h.at[c, sl])


def _edge_es_body(nblk0, nblk1, rpt, ch, table_h, src_h, dst_h, out_h,
                  idx_s, idx_d, rows, tab_sh, acc_sh, *sems):
    c = lax.axis_index("c")
    s = lax.axis_index("s")
    sl = pl.ds(s * rpt, rpt)
    d1 = pltpu.async_copy(table_h.at[sl], tab_sh.at[sl], sems[0])
    d2 = pltpu.async_copy(table_h.at[sl], acc_sh.at[sl], sems[1])
    d1.wait()
    d2.wait()
    plsc.subcore_barrier()

    args = (tab_sh, table_h, src_h, dst_h, idx_s, idx_d,
            rows, acc_sh, sems[:NBUF], sems[NBUF:])

    @pl.when(c == 0)
    def _():
        _edge_loop(ch, nblk0, s * nblk0, *args)

    @pl.when(c == 1)
    def _():
        _edge_loop(ch, nblk1, NS * nblk0 + s * nblk1, *args)

    plsc.subcore_barrier()
    pltpu.sync_copy(acc_sh.at[sl], out_h.at[c, sl])


def _edge_scratch(ch, np_, f):
    return [
        pltpu.VMEM((ch, B), jnp.int32),
        pltpu.VMEM((ch, B), jnp.int32),
        pltpu.VMEM((NBUF, B, f), jnp.float32),
        pltpu.VMEM_SHARED((np_, f), jnp.float32),
        pltpu.VMEM_SHARED((np_, f), jnp.float32),
    ] + [pltpu.SemaphoreType.DMA] * (2 * NBUF)


def _edge_pass_fs(table2, src_r, dst_r, np_, f2, tbt):
    rpt = np_ // NS
    ch = 8
    body = functools.partial(_edge_fs_body, tbt, rpt, ch)
    return pl.kernel(
        body,
        out_type=jax.ShapeDtypeStruct((NC, np_, f2), jnp.float32),
        mesh=_MESH,
        scratch_types=_edge_scratch(ch, np_, f2),
        compiler_params=_SC_PARAMS,
    )(table2, src_r, dst_r)


def _edge_pass_es(table, src_r, dst_r, np_, f, nblk0, nblk1):
    rpt = np_ // NS
    ch = 8
    body = functools.partial(_edge_es_body, nblk0, nblk1, rpt, ch)
    return pl.kernel(
        body,
        out_type=jax.ShapeDtypeStruct((NC, np_, f), jnp.float32),
        mesh=_MESH,
        scratch_types=_edge_scratch(ch, np_, f),
        compiler_params=_SC_PARAMS,
    )(table, src_r, dst_r)


def _b_body(f2, x_ref, w_ref, degp_ref, out_ref):
    deg = degp_ref[0, :] + degp_ref[1, :] + 1.0
    dinv = lax.rsqrt(deg)
    h = jnp.dot(x_ref[...], w_ref[...], preferred_element_type=jnp.float32)
    hp = h * dinv[:, None]
    out_ref[0] = hp[:, :f2]
    out_ref[1] = hp[:, f2:]


def _d_body(acc_ref, degp_ref, b1_ref, w2_ref, out_ref):
    deg = degp_ref[0, :] + degp_ref[1, :] + 1.0
    dinv = lax.rsqrt(deg)
    ssum = jnp.concatenate([acc_ref[0], acc_ref[1]], axis=1)
    a1 = jnp.maximum(ssum * dinv[:, None] + b1_ref[...], 0.0)
    h2 = jnp.dot(a1, w2_ref[...], preferred_element_type=jnp.float32)
    out_ref[...] = h2 * dinv[:, None]


def _f_body(acc_ref, hp_ref, degt_ref, b2_ref, out_ref):
    d = degt_ref[...]
    dinv = lax.rsqrt(d[:, 0] + d[:, 1] + 1.0)
    out_ref[...] = (acc_ref[0] + acc_ref[1] - hp_ref[...]) * dinv[:, None] \
        + b2_ref[...]


def kernel(x, edge_index, W1, b1, W2, b2):
    n, d_in = x.shape
    hid = W1.shape[1]
    n_act = W2.shape[1]
    e = edge_index.shape[1]

    np_ = _ceil_to(n + 1, NS * 40)
    ep = _ceil_to(e, NC * NS * B * 8)
    nblk = ep // (NC * NS * B)
    tbt = ep // (NS * B)
    nblk0 = tbt // 2
    nblk1 = tbt - nblk0
    f2 = hid // NC
    rb = np_ // 16

    pad_idx = jnp.full((2, ep - e), n, dtype=edge_index.dtype)
    ei = jnp.concatenate([edge_index, pad_idx], axis=1)
    src_r = ei[0].reshape(NC * NS * nblk, B)
    dst_r = ei[1].reshape(NC * NS * nblk, B)
    xp = jnp.pad(x, ((0, np_ - n), (0, 0)))

    degp = _degree(dst_r, np_, nblk)

    h1t = pl.pallas_call(
        functools.partial(_b_body, f2),
        grid=(np_ // rb,),
        in_specs=[
            pl.BlockSpec((rb, d_in), lambda i: (i, 0)),
            pl.BlockSpec((d_in, hid), lambda i: (0, 0)),
            pl.BlockSpec((NC, rb), lambda i: (0, i)),
        ],
        out_specs=pl.BlockSpec((NC, rb, f2), lambda i: (0, i, 0)),
        out_shape=jax.ShapeDtypeStruct((NC, np_, f2), jnp.float32),
    )(xp, W1, degp)

    acc1 = _edge_pass_fs(h1t, src_r, dst_r, np_, f2, tbt)

    h2p = pl.pallas_call(
        _d_body,
        grid=(np_ // rb,),
        in_specs=[
            pl.BlockSpec((NC, rb, f2), lambda i: (0, i, 0)),
            pl.BlockSpec((NC, rb), lambda i: (0, i)),
            pl.BlockSpec((1, hid), lambda i: (0, 0)),
            pl.BlockSpec((hid, n_act), lambda i: (0, 0)),
        ],
        out_specs=pl.BlockSpec((rb, n_act), lambda i: (i, 0)),
        out_shape=jax.ShapeDtypeStruct((np_, n_act), jnp.float32),
    )(acc1, degp, b1.reshape(1, hid), W2)

    acc2 = _edge_pass_es(h2p, src_r, dst_r, np_, n_act, nblk0, nblk1)

    rbf = 400
    out = pl.pallas_call(
        _f_body,
        grid=(n // rbf,),
        in_specs=[
            pl.BlockSpec((NC, rbf, n_act), lambda i: (0, i, 0)),
            pl.BlockSpec((rbf, n_act), lambda i: (i, 0)),
            pl.BlockSpec((rbf, NC), lambda i: (i, 0)),
            pl.BlockSpec((1, n_act), lambda i: (0, 0)),
        ],
        out_specs=pl.BlockSpec((rbf, n_act), lambda i: (i, 0)),
        out_shape=jax.ShapeDtypeStruct((n, n_act), jnp.float32),
    )(acc2, h2p, degp.T, b2.reshape(1, n_act))

    return out
